# baseline (speedup 1.0000x reference)
import jax
import jax.numpy as jnp
from jax import lax
from jax.experimental import pallas as pl
from jax.experimental.pallas import tpu as pltpu
from jax.experimental.pallas import tpu_sc as plsc

NC = 2
NS = 16
NW = NC * NS

B = 16384
D = 32
NCHUNK = 1
CB = B // NCHUNK
BPW = CB // NW
K = 16
NG = BPW // K
NP = BPW // 128

BN = 2048


H = K // 2


def _scalars(idx_v, g, ks):
    lane = lax.iota(jnp.int32, 16)
    ids16 = idx_v[pl.ds(g * K, K)]
    return [jnp.sum(jnp.where(lane == k, ids16, 0)) for k in ks]


def _fire(tab_hbm, gbuf, sem, vs, slot0):
    for k, v in enumerate(vs):
        col0 = pl.multiple_of((v >> 7) << 7, 128)
        pltpu.async_copy(tab_hbm.at[:, pl.ds(col0, 128)],
                         gbuf.at[slot0 + k], sem)


def _drain(tab_hbm, gbuf, sem, slot0):
    for k in range(H):
        pltpu.make_async_copy(tab_hbm.at[:, pl.ds(0, 128)],
                              gbuf.at[slot0 + k], sem).wait()


def _extract(gbuf, cols, vs, g, slot0):
    lane = lax.iota(jnp.int32, 16)
    f_lo = lane
    f_hi = lane + 16
    for k, v in enumerate(vs):
        i = g * K + slot0 + k
        c16 = jnp.full((16,), v & 127, jnp.int32)
        l16 = jnp.full((16,), i & 127, jnp.int32)
        lo = plsc.load_gather(gbuf.at[slot0 + k], [f_lo, c16])
        hi = plsc.load_gather(gbuf.at[slot0 + k], [f_hi, c16])
        piece = cols.at[i >> 7]
        plsc.store_scatter(piece, [f_lo, l16], lo)
        plsc.store_scatter(piece, [f_hi, l16], hi)


def _gather_table(tab_hbm, idx_v, gbuf, cols, sem_a, sem_b):
    _fire(tab_hbm, gbuf, sem_a, _scalars(idx_v, 0, range(H)), 0)

    def body(g, carry):
        vs_a = _scalars(idx_v, g, range(H))
        vs_b = _scalars(idx_v, g, range(H, K))
        _fire(tab_hbm, gbuf, sem_b, vs_b, H)
        _drain(tab_hbm, gbuf, sem_a, 0)
        _extract(gbuf, cols, vs_a, g, 0)
        gn = jnp.minimum(g + 1, NG - 1)
        vs_an = _scalars(idx_v, gn, range(H))
        _fire(tab_hbm, gbuf, sem_a, vs_an, 0)
        _drain(tab_hbm, gbuf, sem_b, H)
        _extract(gbuf, cols, vs_b, g, H)
        return carry

    lax.fori_loop(0, NG, body, 0)
    _drain(tab_hbm, gbuf, sem_a, 0)


def _gather_body(pid_hbm, ptab_hbm, pout_hbm,
                 pidx_v, gbuf, pcols, sem_a, sem_b):
    wid = lax.axis_index("s") * NC + lax.axis_index("c")
    base = wid * BPW
    pltpu.sync_copy(pid_hbm.at[pl.ds(base, BPW)], pidx_v)

    _gather_table(ptab_hbm, pidx_v, gbuf, pcols, sem_a, sem_b)

    for j in range(NP):
        pltpu.sync_copy(pcols.at[j], pout_hbm.at[:, pl.ds(base + j * 128, 128)])


def _sc_gather_prod(pid, ptab_t):
    mesh = plsc.VectorSubcoreMesh(
        core_axis_name="c", subcore_axis_name="s",
        num_cores=NC, num_subcores=NS)
    f = pl.kernel(
        _gather_body,
        out_type=jax.ShapeDtypeStruct((D, CB), jnp.float32),
        mesh=mesh,
        scratch_types=[
            pltpu.VMEM((BPW,), jnp.int32),
            pltpu.VMEM((K, D, 128), jnp.float32),
            pltpu.VMEM((NP, D, 128), jnp.float32),
            pltpu.SemaphoreType.DMA,
            pltpu.SemaphoreType.DMA,
        ],
        compiler_params=pltpu.CompilerParams(needs_layout_passes=False),
    )
    return f(pid, ptab_t)


UCH = 128
UG = BPW // UCH


def _ugather_body(uid_hbm, utab_hbm, uout_hbm, uidx_v, urows_v, sem):
    wid = lax.axis_index("s") * NC + lax.axis_index("c")
    base = wid * BPW
    pltpu.sync_copy(uid_hbm.at[wid], uidx_v)
    copies = [
        pltpu.async_copy(utab_hbm.at[uidx_v.at[j]],
                         urows_v.at[pl.ds(j * UCH, UCH)], sem)
        for j in range(UG)
    ]
    for c in copies:
        c.wait()
    pltpu.sync_copy(urows_v, uout_hbm.at[pl.ds(base, BPW)])


def _sc_gather_user(uid3, utab):
    mesh = plsc.VectorSubcoreMesh(
        core_axis_name="c", subcore_axis_name="s",
        num_cores=NC, num_subcores=NS)
    f = pl.kernel(
        _ugather_body,
        out_type=jax.ShapeDtypeStruct((CB, D), jnp.float32),
        mesh=mesh,
        scratch_types=[
            pltpu.VMEM((UG, UCH), jnp.int32),
            pltpu.VMEM((BPW, D), jnp.float32),
            pltpu.SemaphoreType.DMA,
        ],
        compiler_params=pltpu.CompilerParams(use_tc_tiling_on_sc=False),
    )
    return f(uid3, utab)


def _mlp_body(u_ref, p_ref, w1u_ref, w1p_ref, b1_ref, w2_ref, b2_ref,
              w3_ref, b3_ref, out_ref):
    cdn = (((0,), (0,)), ((), ()))
    cdn_u = (((0,), (1,)), ((), ()))
    h = (lax.dot_general(w1u_ref[...], u_ref[...], cdn_u,
                         preferred_element_type=jnp.float32)
         + lax.dot_general(w1p_ref[...], p_ref[...], cdn,
                           preferred_element_type=jnp.float32)
         + b1_ref[...])
    h = jnp.maximum(h, 0.0)
    h = jnp.maximum(lax.dot_general(w2_ref[...], h, cdn,
                                    preferred_element_type=jnp.float32)
                    + b2_ref[...], 0.0)
    s = lax.dot_general(w3_ref[...], h, cdn,
                        preferred_element_type=jnp.float32) + b3_ref[...]
    out_ref[...] = s.reshape(BN)


def _mlp(u_t, p_t, w1u, w1p, b1c, w2, b2c, w3, b3c):
    n_blocks = CB // BN
    full = lambda i: (0, 0)
    return pl.pallas_call(
        _mlp_body,
        grid=(n_blocks,),
        in_specs=[
            pl.BlockSpec((BN, D), lambda i: (i, 0)),
            pl.BlockSpec((D, BN), lambda i: (0, i)),
            pl.BlockSpec(w1u.shape, full),
            pl.BlockSpec(w1p.shape, full),
            pl.BlockSpec(b1c.shape, full),
            pl.BlockSpec(w2.shape, full),
            pl.BlockSpec(b2c.shape, full),
            pl.BlockSpec(w3.shape, full),
            pl.BlockSpec(b3c.shape, full),
        ],
        out_specs=pl.BlockSpec((BN,), lambda i: (i,)),
        out_shape=jax.ShapeDtypeStruct((CB,), jnp.float32),
        compiler_params=pltpu.CompilerParams(
            dimension_semantics=("arbitrary",)),
    )(u_t, p_t, w1u, w1p, b1c, w2, b2c, w3, b3c)


@jax.jit
def kernel(user_id, product_id, user_table, product_table, W1, b1, W2, b2,
           W3, b3):
    uid3 = user_id.astype(jnp.int32).reshape(NW, UG, UCH)
    pid = product_id.astype(jnp.int32)
    p_t = _sc_gather_prod(pid, product_table.T)
    u_emb = _sc_gather_user(uid3, user_table)
    out = _mlp(u_emb, p_t, W1[:D], W1[D:], b1.reshape(-1, 1),
               W2, b2.reshape(-1, 1), W3, b3.reshape(1, 1))
    return out.reshape(B, 1)


# speedup vs baseline: 1.1103x; 1.1103x over previous
import jax
import jax.numpy as jnp
from jax import lax
from jax.experimental import pallas as pl
from jax.experimental.pallas import tpu as pltpu
from jax.experimental.pallas import tpu_sc as plsc

NC = 2
NS = 16
NW = NC * NS

B = 16384
D = 32
NCHUNK = 1
CB = B // NCHUNK
BPW = CB // NW
K = 16
NG = BPW // K
NP = BPW // 128

BN = 2048


H = K // 2


def _scalars(idx_v, g, ks):
    lane = lax.iota(jnp.int32, 16)
    ids16 = idx_v[pl.ds(g * K, K)]
    return [jnp.sum(jnp.where(lane == k, ids16, 0)) for k in ks]


def _fire(tab_hbm, gbuf, sem, vs, slot0):
    for k, v in enumerate(vs):
        col0 = pl.multiple_of((v >> 7) << 7, 128)
        pltpu.async_copy(tab_hbm.at[:, pl.ds(col0, 128)],
                         gbuf.at[slot0 + k], sem)


def _drain(tab_hbm, gbuf, sem, slot0):
    for k in range(H):
        pltpu.make_async_copy(tab_hbm.at[:, pl.ds(0, 128)],
                              gbuf.at[slot0 + k], sem).wait()


def _extract(gbuf, cols, vs, g, slot0):
    lane = lax.iota(jnp.int32, 16)
    f_lo = lane
    f_hi = lane + 16
    for k, v in enumerate(vs):
        i = g * K + slot0 + k
        c16 = jnp.full((16,), v & 127, jnp.int32)
        l16 = jnp.full((16,), i & 127, jnp.int32)
        lo = plsc.load_gather(gbuf.at[slot0 + k], [f_lo, c16])
        hi = plsc.load_gather(gbuf.at[slot0 + k], [f_hi, c16])
        piece = cols.at[i >> 7]
        plsc.store_scatter(piece, [f_lo, l16], lo)
        plsc.store_scatter(piece, [f_hi, l16], hi)


def _gather_table(tab_hbm, idx_v, gbuf, cols, sem_a, sem_b):
    _fire(tab_hbm, gbuf, sem_a, _scalars(idx_v, 0, range(H)), 0)

    def body(g, carry):
        vs_a = _scalars(idx_v, g, range(H))
        vs_b = _scalars(idx_v, g, range(H, K))
        _fire(tab_hbm, gbuf, sem_b, vs_b, H)
        _drain(tab_hbm, gbuf, sem_a, 0)
        _extract(gbuf, cols, vs_a, g, 0)
        gn = jnp.minimum(g + 1, NG - 1)
        vs_an = _scalars(idx_v, gn, range(H))
        _fire(tab_hbm, gbuf, sem_a, vs_an, 0)
        _drain(tab_hbm, gbuf, sem_b, H)
        _extract(gbuf, cols, vs_b, g, H)
        return carry

    lax.fori_loop(0, NG, body, 0)
    _drain(tab_hbm, gbuf, sem_a, 0)


def _gather_body(pid_hbm, ptab_hbm, pout_hbm,
                 pidx_v, gbuf, pcols, sem_a, sem_b):
    wid = lax.axis_index("s") * NC + lax.axis_index("c")
    base = wid * BPW
    pltpu.sync_copy(pid_hbm.at[pl.ds(base, BPW)], pidx_v)

    _gather_table(ptab_hbm, pidx_v, gbuf, pcols, sem_a, sem_b)

    for j in range(NP):
        pltpu.sync_copy(pcols.at[j], pout_hbm.at[:, pl.ds(base + j * 128, 128)])


def _sc_gather_prod(pid, ptab_t):
    mesh = plsc.VectorSubcoreMesh(
        core_axis_name="c", subcore_axis_name="s",
        num_cores=NC, num_subcores=NS)
    f = pl.kernel(
        _gather_body,
        out_type=jax.ShapeDtypeStruct((D, CB), jnp.float32),
        mesh=mesh,
        scratch_types=[
            pltpu.VMEM((BPW,), jnp.int32),
            pltpu.VMEM((K, D, 128), jnp.float32),
            pltpu.VMEM((NP, D, 128), jnp.float32),
            pltpu.SemaphoreType.DMA,
            pltpu.SemaphoreType.DMA,
        ],
        compiler_params=pltpu.CompilerParams(needs_layout_passes=False),
    )
    return f(pid, ptab_t)


UCH = 128
UG = BPW // UCH


def _ugather_body(uid_hbm, utab_hbm, uout_hbm, uidx_v, urows_v, sem):
    wid = lax.axis_index("s") * NC + lax.axis_index("c")
    base = wid * BPW
    pltpu.sync_copy(uid_hbm.at[wid], uidx_v)
    copies = [
        pltpu.async_copy(utab_hbm.at[uidx_v.at[j]],
                         urows_v.at[pl.ds(j * UCH, UCH)], sem)
        for j in range(UG)
    ]
    for c in copies:
        c.wait()
    pltpu.sync_copy(urows_v, uout_hbm.at[pl.ds(base, BPW)])


def _sc_gather_user(uid3, utab):
    mesh = plsc.VectorSubcoreMesh(
        core_axis_name="c", subcore_axis_name="s",
        num_cores=NC, num_subcores=NS)
    f = pl.kernel(
        _ugather_body,
        out_type=jax.ShapeDtypeStruct((CB, D), jnp.float32),
        mesh=mesh,
        scratch_types=[
            pltpu.VMEM((UG, UCH), jnp.int32),
            pltpu.VMEM((BPW, D), jnp.float32),
            pltpu.SemaphoreType.DMA,
        ],
        compiler_params=pltpu.CompilerParams(use_tc_tiling_on_sc=False),
    )
    return f(uid3, utab)


def _mlp_body(u_ref, p_ref, w1u_ref, w1p_ref, b1_ref, w2_ref, b2_ref,
              w3_ref, b3_ref, out_ref):
    cdn = (((0,), (0,)), ((), ()))
    cdn_u = (((0,), (1,)), ((), ()))
    h = (lax.dot_general(w1u_ref[...], u_ref[...], cdn_u,
                         preferred_element_type=jnp.float32)
         + lax.dot_general(w1p_ref[...], p_ref[...], cdn,
                           preferred_element_type=jnp.float32)
         + b1_ref[...])
    h = jnp.maximum(h, 0.0)
    h = jnp.maximum(lax.dot_general(w2_ref[...], h, cdn,
                                    preferred_element_type=jnp.float32)
                    + b2_ref[...], 0.0)
    s = lax.dot_general(w3_ref[...], h, cdn,
                        preferred_element_type=jnp.float32) + b3_ref[...]
    out_ref[...] = s.reshape(BN)


def _mlp(u_t, p_t, w1u, w1p, b1c, w2, b2c, w3, b3c):
    n_blocks = CB // BN
    full = lambda i: (0, 0)
    return pl.pallas_call(
        _mlp_body,
        grid=(n_blocks,),
        in_specs=[
            pl.BlockSpec((BN, D), lambda i: (i, 0)),
            pl.BlockSpec((D, BN), lambda i: (0, i)),
            pl.BlockSpec(w1u.shape, full),
            pl.BlockSpec(w1p.shape, full),
            pl.BlockSpec(b1c.shape, full),
            pl.BlockSpec(w2.shape, full),
            pl.BlockSpec(b2c.shape, full),
            pl.BlockSpec(w3.shape, full),
            pl.BlockSpec(b3c.shape, full),
        ],
        out_specs=pl.BlockSpec((BN,), lambda i: (i,)),
        out_shape=jax.ShapeDtypeStruct((CB,), jnp.float32),
        compiler_params=pltpu.CompilerParams(
            dimension_semantics=("arbitrary",)),
    )(u_t, p_t, w1u, w1p, b1c, w2, b2c, w3, b3c)


@jax.jit
def kernel(user_id, product_id, user_table, product_table, W1, b1, W2, b2,
           W3, b3):
    uid3 = user_id.astype(jnp.int32).reshape(NW, UG, UCH)
    pid = product_id.astype(jnp.int32)
    p_t = _sc_gather_prod(pid, product_table.T)
    uid3, p_t = lax.optimization_barrier((uid3, p_t))
    u_emb = _sc_gather_user(uid3, user_table)
    out = _mlp(u_emb, p_t, W1[:D], W1[D:], b1.reshape(-1, 1),
               W2, b2.reshape(-1, 1), W3, b3.reshape(1, 1))
    return out.reshape(B, 1)
